# baseline (speedup 1.0000x reference)
import functools

import jax
import jax.numpy as jnp
from jax import lax
from jax.experimental import pallas as pl
from jax.experimental.pallas import tpu as pltpu
from jax.experimental.pallas import tpu_sc as plsc

N = 10000
D = 128
H = 32
C = 10
E = 320000

NC = 2
NS = 16
NW = NC * NS

CHUNK = 128
NCHUNK = 80
E_PAD = NW * NCHUNK * CHUNK
N_PAD = 10240
LPT = N_PAD // NS
W16 = 16
P8 = N_PAD // 8
PPT = P8 // NS
NV = N // 8


def _dotT(a, w):
    return lax.dot_general(a, w, (((1,), (1,)), ((), ())),
                           preferred_element_type=jnp.float32)


def _dot(a, w):
    return lax.dot_general(a, w, (((1,), (0,)), ((), ())),
                           preferred_element_type=jnp.float32)


def _lin1_body(x8_ref, wlo_ref, whi_ref, wrlo_ref, wrhi_ref,
               plo_ref, phi_ref, rlo_ref, rhi_ref):
    x8 = x8_ref[...]
    plo_ref[...] = _dot(x8, wlo_ref[...])
    phi_ref[...] = _dot(x8, whi_ref[...])
    rlo_ref[...] = _dot(x8, wrlo_ref[...])
    rhi_ref[...] = _dot(x8, wrhi_ref[...])


def _mid_body(plo_ref, phi_ref, rlo_ref, rhi_ref, b1lo_ref, b1hi_ref,
              wlo_ref, whi_ref, wrlo_ref, wrhi_ref, q_ref, s_ref):
    row = lax.broadcasted_iota(jnp.int32, (P8, 128), 0)
    valid = row < NV
    hlo = jnp.maximum(plo_ref[0] + plo_ref[1] + b1lo_ref[...] + rlo_ref[...], 0.0)
    hhi = jnp.maximum(phi_ref[0] + phi_ref[1] + b1hi_ref[...] + rhi_ref[...], 0.0)
    hlo = jnp.where(valid, hlo, 0.0)
    hhi = jnp.where(valid, hhi, 0.0)
    q_ref[...] = _dot(hlo, wlo_ref[...]) + _dot(hhi, whi_ref[...])
    s_ref[...] = _dot(hlo, wrlo_ref[...]) + _dot(hhi, wrhi_ref[...])


def _loss_body(parts_ref, s_ref, b2_ref, y_ref, out_ref):
    lg = parts_ref[0] + parts_ref[1] + s_ref[...] + b2_ref[...]
    lane = lax.broadcasted_iota(jnp.int32, lg.shape, 1)
    col = lane % W16
    lg = jnp.where(col < C, lg, -1e30)
    v = lg
    for k in (8, 4, 2, 1):
        shifted = jnp.concatenate(
            [v[:, k:], jnp.full((P8, k), -1e30, jnp.float32)], axis=1)
        v = jnp.maximum(v, shifted)
    lrow = lax.broadcasted_iota(jnp.int32, (128, 128), 0)
    lcol = lax.broadcasted_iota(jnp.int32, (128, 128), 1)
    B = jnp.where((lrow // W16 == lcol // W16) & (lrow % W16 == 0), 1.0, 0.0)
    m = _dot(jnp.where(col == 0, v, 0.0), B.astype(jnp.float32))
    e = jnp.exp(lg - m)
    srow = lax.broadcasted_iota(jnp.int32, (128, 8), 0)
    scol = lax.broadcasted_iota(jnp.int32, (128, 8), 1)
    S = jnp.where(srow // W16 == scol, 1.0, 0.0).astype(jnp.float32)
    se = _dot(e, S)
    lse_m = jnp.log(se)
    pick = jnp.where(col == y_ref[...], lg - m, 0.0)
    picked_m = _dot(pick, S)
    nll = lse_m - picked_m
    rowi = lax.broadcasted_iota(jnp.int32, nll.shape, 0)
    nll = jnp.where(rowi < NV, nll, 0.0)
    out_ref[...] = (jnp.sum(nll) / jnp.float32(N)).reshape(1, 1)


def _make_seg_sum():
    mesh = plsc.VectorSubcoreMesh(core_axis_name="c", subcore_axis_name="s",
                                  num_cores=NC, num_subcores=NS)

    @functools.partial(
        pl.kernel,
        out_type=jax.ShapeDtypeStruct((NC, P8, 128), jnp.float32),
        mesh=mesh,
        scratch_types=[
            pltpu.VMEM((NCHUNK * CHUNK,), jnp.int32),
            pltpu.VMEM((NCHUNK, CHUNK), jnp.int32),
            *[pltpu.VMEM((CHUNK, W16), jnp.float32) for _ in range(8)],
            pltpu.VMEM((PPT, 128), jnp.float32),
            pltpu.VMEM((LPT, W16), jnp.float32),
            pltpu.VMEM_SHARED((N_PAD, W16), jnp.float32),
            pltpu.VMEM_SHARED((N_PAD, W16), jnp.float32),
            pltpu.SemaphoreType.DMA,
            pltpu.SemaphoreType.DMA,
        ],
        compiler_params=pltpu.CompilerParams(use_tc_tiling_on_sc=False),
    )
    def seg(src_hbm, dst_hbm, table_hbm, out_hbm,
            src_v, dst_v, r0, r1, r2, r3, r4, r5, r6, r7, pbig, rbig,
            tab, acc, sem_g, sem_s):
        c = lax.axis_index("c")
        s = lax.axis_index("s")
        w = c * NS + s
        rows_a = [r0, r1, r2, r3]
        rows_b = [r4, r5, r6, r7]

        epw = NCHUNK * CHUNK
        tail = E - (NW - 1) * epw

        idx_cp = pltpu.async_copy(dst_hbm.at[w], dst_v, sem_s)

        @pl.when(w < NW - 1)
        def _():
            pltpu.sync_copy(src_hbm.at[pl.ds(w * epw, epw)], src_v)

        @pl.when(w == NW - 1)
        def _():
            pltpu.sync_copy(src_hbm.at[pl.ds((NW - 1) * epw, tail)],
                            src_v.at[pl.ds(0, tail)])

            def pfill(i, carry):
                vals = N + (lax.iota(jnp.int32, 16) + 16 * i) % (N_PAD - N)
                src_v[pl.ds(tail + 16 * i, 16)] = vals
                return carry

            lax.fori_loop(0, (epw - tail) // 16, pfill, 0)

        idx_cp.wait()

        def zstore(i, carry):
            rbig[i, pl.ds(0, 16)] = jnp.zeros((16,), jnp.float32)
            return carry

        lax.fori_loop(0, LPT, zstore, 0)
        pltpu.sync_copy(rbig, acc.at[pl.ds(s * LPT, LPT)])

        pltpu.sync_copy(table_hbm.at[pl.ds(s * PPT, PPT)], pbig)

        def rbody(i, carry):
            for kk in range(8):
                rbig[8 * i + kk, pl.ds(0, 16)] = pbig[i, pl.ds(16 * kk, 16)]
            return carry

        lax.fori_loop(0, PPT, rbody, 0)
        pltpu.sync_copy(rbig, tab.at[pl.ds(s * LPT, LPT)])
        plsc.subcore_barrier()

        def gather(j, buf):
            pltpu.async_copy(tab.at[src_v.at[pl.ds(j * CHUNK, CHUNK)]],
                             buf, sem_g)

        def gather_wait(j, buf):
            pltpu.make_async_copy(tab.at[src_v.at[pl.ds(j * CHUNK, CHUNK)]],
                                  buf, sem_g).wait()

        def scat(j, buf):
            pltpu.async_copy(buf, acc.at[dst_v.at[j]], sem_s, add=True)

        def scat_wait(j, buf):
            pltpu.make_async_copy(buf, acc.at[dst_v.at[j]], sem_s).wait()

        for b in range(4):
            gather(b, rows_a[b])

        npair = NCHUNK // 8

        def body(k, carry):
            base = 8 * k
            for b in range(4):
                gather_wait(base + b, rows_a[b])
                scat(base + b, rows_a[b])
            for b in range(4):
                gather(base + 4 + b, rows_b[b])
            for b in range(4):
                gather_wait(base + 4 + b, rows_b[b])
                scat(base + 4 + b, rows_b[b])
            for b in range(4):
                scat_wait(base + b, rows_a[b])

            @pl.when(k < npair - 1)
            def _():
                for b in range(4):
                    gather(base + 8 + b, rows_a[b])

            for b in range(4):
                scat_wait(base + 4 + b, rows_b[b])
            return carry

        lax.fori_loop(0, npair, body, 0)
        plsc.subcore_barrier()

        pltpu.sync_copy(acc.at[pl.ds(s * LPT, LPT)], rbig)

        def wbody(i, carry):
            for kk in range(8):
                pbig[i, pl.ds(16 * kk, 16)] = rbig[8 * i + kk, pl.ds(0, 16)]
            return carry

        lax.fori_loop(0, PPT, wbody, 0)
        pltpu.sync_copy(pbig, out_hbm.at[c].at[pl.ds(s * PPT, PPT)])

    return seg


_seg_sum_cache = {}


def _seg_sum():
    if "k" not in _seg_sum_cache:
        _seg_sum_cache["k"] = _make_seg_sum()
    return _seg_sum_cache["k"]


def kernel(x, edge_index, y, W1_rel, b1_rel, W1_root, W2_rel, b2_rel, W2_root):
    f32 = jnp.float32

    src = edge_index[0]
    pad_rows = N + (jnp.arange(E_PAD - E, dtype=jnp.int32) % (N_PAD - N))
    dst = jnp.concatenate([edge_index[1], pad_rows]).reshape(NW, NCHUNK, CHUNK)
    x8 = jnp.pad(x, ((0, N_PAD - N), (0, 0))).reshape(P8, 8 * D)
    eye8f = jnp.eye(8, dtype=f32)
    w1lo = jnp.kron(eye8f, W1_rel.T[:, :W16])
    w1hi = jnp.kron(eye8f, W1_rel.T[:, W16:])
    w1rlo = jnp.kron(eye8f, W1_root.T[:, :W16])
    w1rhi = jnp.kron(eye8f, W1_root.T[:, W16:])
    w2rel_p = jnp.zeros((W16, H), f32).at[:C].set(W2_rel)
    w2root_p = jnp.zeros((W16, H), f32).at[:C].set(W2_root)
    eye8 = jnp.eye(8, dtype=f32)
    wlo = jnp.kron(eye8, w2rel_p.T[:W16])
    whi = jnp.kron(eye8, w2rel_p.T[W16:])
    wrlo = jnp.kron(eye8, w2root_p.T[:W16])
    wrhi = jnp.kron(eye8, w2root_p.T[W16:])
    b1lo = jnp.tile(b1_rel[:W16], 8).reshape(1, 128)
    b1hi = jnp.tile(b1_rel[W16:], 8).reshape(1, 128)
    b2_8 = jnp.tile(jnp.zeros((W16,), f32).at[:C].set(b2_rel), 8).reshape(1, 128)
    y_pad = jnp.pad(y.astype(jnp.int32), (0, N_PAD - N)).reshape(P8, 8)
    y_exp = jnp.repeat(y_pad, W16, axis=1)

    plo, phi, rlo, rhi = pl.pallas_call(
        _lin1_body,
        out_shape=[jax.ShapeDtypeStruct((P8, 128), f32)] * 4,
    )(x8, w1lo, w1hi, w1rlo, w1rhi)

    parts_lo = _seg_sum()(src, dst, plo)
    parts_hi = _seg_sum()(src, dst, phi)

    q8, s8 = pl.pallas_call(
        _mid_body,
        out_shape=[jax.ShapeDtypeStruct((P8, 128), f32)] * 2,
    )(parts_lo, parts_hi, rlo, rhi, b1lo, b1hi, wlo, whi, wrlo, wrhi)

    parts2 = _seg_sum()(src, dst, q8)

    out = pl.pallas_call(
        _loss_body,
        out_shape=jax.ShapeDtypeStruct((1, 1), f32),
    )(parts2, s8, b2_8, y_exp)

    return (out[0, 0],)


# speedup vs baseline: 1.0315x; 1.0315x over previous
import functools

import jax
import jax.numpy as jnp
from jax import lax
from jax.experimental import pallas as pl
from jax.experimental.pallas import tpu as pltpu
from jax.experimental.pallas import tpu_sc as plsc

N = 10000
D = 128
H = 32
C = 10
E = 320000

NC = 2
NS = 16
NW = NC * NS

CHUNK = 128
NCHUNK = 80
E_PAD = NW * NCHUNK * CHUNK
N_PAD = 10240
LPT = N_PAD // NS
W16 = 16
P8 = N_PAD // 8
PPT = P8 // NS
NV = N // 8


def _dotT(a, w):
    return lax.dot_general(a, w, (((1,), (1,)), ((), ())),
                           preferred_element_type=jnp.float32)


def _dot(a, w):
    return lax.dot_general(a, w, (((1,), (0,)), ((), ())),
                           preferred_element_type=jnp.float32)


def _lin1_body(x8_ref, wlo_ref, whi_ref, wrlo_ref, wrhi_ref,
               plo_ref, phi_ref, rlo_ref, rhi_ref):
    x8 = x8_ref[...]
    plo_ref[...] = _dot(x8, wlo_ref[...])
    phi_ref[...] = _dot(x8, whi_ref[...])
    rlo_ref[...] = _dot(x8, wrlo_ref[...])
    rhi_ref[...] = _dot(x8, wrhi_ref[...])


def _mid_body(plo_ref, phi_ref, rlo_ref, rhi_ref, b1lo_ref, b1hi_ref,
              wlo_ref, whi_ref, wrlo_ref, wrhi_ref, q_ref, s_ref):
    row = lax.broadcasted_iota(jnp.int32, (P8, 128), 0)
    valid = row < NV
    hlo = jnp.maximum(plo_ref[0] + plo_ref[1] + b1lo_ref[...] + rlo_ref[...], 0.0)
    hhi = jnp.maximum(phi_ref[0] + phi_ref[1] + b1hi_ref[...] + rhi_ref[...], 0.0)
    hlo = jnp.where(valid, hlo, 0.0)
    hhi = jnp.where(valid, hhi, 0.0)
    q_ref[...] = _dot(hlo, wlo_ref[...]) + _dot(hhi, whi_ref[...])
    s_ref[...] = _dot(hlo, wrlo_ref[...]) + _dot(hhi, wrhi_ref[...])


def _loss_body(parts_ref, s_ref, b2_ref, y_ref, out_ref):
    lg = parts_ref[0] + parts_ref[1] + s_ref[...] + b2_ref[...]
    lane = lax.broadcasted_iota(jnp.int32, lg.shape, 1)
    col = lane % W16
    lg = jnp.where(col < C, lg, -1e30)
    v = lg
    for k in (8, 4, 2, 1):
        shifted = jnp.concatenate(
            [v[:, k:], jnp.full((P8, k), -1e30, jnp.float32)], axis=1)
        v = jnp.maximum(v, shifted)
    lrow = lax.broadcasted_iota(jnp.int32, (128, 128), 0)
    lcol = lax.broadcasted_iota(jnp.int32, (128, 128), 1)
    B = jnp.where((lrow // W16 == lcol // W16) & (lrow % W16 == 0), 1.0, 0.0)
    m = _dot(jnp.where(col == 0, v, 0.0), B.astype(jnp.float32))
    e = jnp.exp(lg - m)
    srow = lax.broadcasted_iota(jnp.int32, (128, 8), 0)
    scol = lax.broadcasted_iota(jnp.int32, (128, 8), 1)
    S = jnp.where(srow // W16 == scol, 1.0, 0.0).astype(jnp.float32)
    se = _dot(e, S)
    lse_m = jnp.log(se)
    pick = jnp.where(col == y_ref[...], lg - m, 0.0)
    picked_m = _dot(pick, S)
    nll = lse_m - picked_m
    rowi = lax.broadcasted_iota(jnp.int32, nll.shape, 0)
    nll = jnp.where(rowi < NV, nll, 0.0)
    out_ref[...] = (jnp.sum(nll) / jnp.float32(N)).reshape(1, 1)


def _make_seg_sum():
    mesh = plsc.VectorSubcoreMesh(core_axis_name="c", subcore_axis_name="s",
                                  num_cores=NC, num_subcores=NS)

    @functools.partial(
        pl.kernel,
        out_type=jax.ShapeDtypeStruct((NC, P8, 128), jnp.float32),
        mesh=mesh,
        scratch_types=[
            pltpu.VMEM((NCHUNK, CHUNK), jnp.int32),
            pltpu.VMEM((NCHUNK, CHUNK), jnp.int32),
            *[pltpu.VMEM((CHUNK, W16), jnp.float32) for _ in range(8)],
            pltpu.VMEM((PPT, 128), jnp.float32),
            pltpu.VMEM((LPT, W16), jnp.float32),
            pltpu.VMEM_SHARED((N_PAD, W16), jnp.float32),
            pltpu.VMEM_SHARED((N_PAD, W16), jnp.float32),
            pltpu.SemaphoreType.DMA,
            pltpu.SemaphoreType.DMA,
        ],
        compiler_params=pltpu.CompilerParams(use_tc_tiling_on_sc=False),
    )
    def seg(src_hbm, dst_hbm, table_hbm, out_hbm,
            src_v, dst_v, r0, r1, r2, r3, r4, r5, r6, r7, pbig, rbig,
            tab, acc, sem_g, sem_s):
        c = lax.axis_index("c")
        s = lax.axis_index("s")
        w = c * NS + s
        rows_a = [r0, r1, r2, r3]
        rows_b = [r4, r5, r6, r7]

        idx_cp = pltpu.async_copy(src_hbm.at[w], src_v, sem_s)
        pltpu.sync_copy(dst_hbm.at[w], dst_v)

        def zstore(i, carry):
            rbig[i, pl.ds(0, 16)] = jnp.zeros((16,), jnp.float32)
            return carry

        lax.fori_loop(0, LPT, zstore, 0)
        pltpu.sync_copy(rbig, acc.at[pl.ds(s * LPT, LPT)])

        pltpu.sync_copy(table_hbm.at[pl.ds(s * PPT, PPT)], pbig)

        def rbody(i, carry):
            for kk in range(8):
                rbig[8 * i + kk, pl.ds(0, 16)] = pbig[i, pl.ds(16 * kk, 16)]
            return carry

        lax.fori_loop(0, PPT, rbody, 0)
        pltpu.sync_copy(rbig, tab.at[pl.ds(s * LPT, LPT)])
        idx_cp.wait()
        plsc.subcore_barrier()

        def gather(j, buf):
            pltpu.async_copy(tab.at[src_v.at[j]], buf, sem_g)

        def gather_wait(j, buf):
            pltpu.make_async_copy(tab.at[src_v.at[j]], buf, sem_g).wait()

        def scat(j, buf):
            pltpu.async_copy(buf, acc.at[dst_v.at[j]], sem_s, add=True)

        def scat_wait(j, buf):
            pltpu.make_async_copy(buf, acc.at[dst_v.at[j]], sem_s).wait()

        for b in range(4):
            gather(b, rows_a[b])

        npair = NCHUNK // 8

        def body(k, carry):
            base = 8 * k
            for b in range(4):
                gather_wait(base + b, rows_a[b])
                scat(base + b, rows_a[b])
            for b in range(4):
                gather(base + 4 + b, rows_b[b])
            for b in range(4):
                gather_wait(base + 4 + b, rows_b[b])
                scat(base + 4 + b, rows_b[b])
            for b in range(4):
                scat_wait(base + b, rows_a[b])

            @pl.when(k < npair - 1)
            def _():
                for b in range(4):
                    gather(base + 8 + b, rows_a[b])

            for b in range(4):
                scat_wait(base + 4 + b, rows_b[b])
            return carry

        lax.fori_loop(0, npair, body, 0)
        plsc.subcore_barrier()

        pltpu.sync_copy(acc.at[pl.ds(s * LPT, LPT)], rbig)

        def wbody(i, carry):
            for kk in range(8):
                pbig[i, pl.ds(16 * kk, 16)] = rbig[8 * i + kk, pl.ds(0, 16)]
            return carry

        lax.fori_loop(0, PPT, wbody, 0)
        pltpu.sync_copy(pbig, out_hbm.at[c].at[pl.ds(s * PPT, PPT)])

    return seg


_seg_sum_cache = {}


def _seg_sum():
    if "k" not in _seg_sum_cache:
        _seg_sum_cache["k"] = _make_seg_sum()
    return _seg_sum_cache["k"]


def kernel(x, edge_index, y, W1_rel, b1_rel, W1_root, W2_rel, b2_rel, W2_root):
    f32 = jnp.float32

    pad_rows = N + (jnp.arange(E_PAD - E, dtype=jnp.int32) % (N_PAD - N))
    src = jnp.concatenate([edge_index[0], pad_rows]).reshape(NW, NCHUNK, CHUNK)
    dst = jnp.concatenate([edge_index[1], pad_rows]).reshape(NW, NCHUNK, CHUNK)
    x8 = jnp.pad(x, ((0, N_PAD - N), (0, 0))).reshape(P8, 8 * D)
    eye8f = jnp.eye(8, dtype=f32)
    w1lo = jnp.kron(eye8f, W1_rel.T[:, :W16])
    w1hi = jnp.kron(eye8f, W1_rel.T[:, W16:])
    w1rlo = jnp.kron(eye8f, W1_root.T[:, :W16])
    w1rhi = jnp.kron(eye8f, W1_root.T[:, W16:])
    w2rel_p = jnp.zeros((W16, H), f32).at[:C].set(W2_rel)
    w2root_p = jnp.zeros((W16, H), f32).at[:C].set(W2_root)
    eye8 = jnp.eye(8, dtype=f32)
    wlo = jnp.kron(eye8, w2rel_p.T[:W16])
    whi = jnp.kron(eye8, w2rel_p.T[W16:])
    wrlo = jnp.kron(eye8, w2root_p.T[:W16])
    wrhi = jnp.kron(eye8, w2root_p.T[W16:])
    b1lo = jnp.tile(b1_rel[:W16], 8).reshape(1, 128)
    b1hi = jnp.tile(b1_rel[W16:], 8).reshape(1, 128)
    b2_8 = jnp.tile(jnp.zeros((W16,), f32).at[:C].set(b2_rel), 8).reshape(1, 128)
    y_pad = jnp.pad(y.astype(jnp.int32), (0, N_PAD - N)).reshape(P8, 8)
    y_exp = jnp.repeat(y_pad, W16, axis=1)

    plo, phi, rlo, rhi = pl.pallas_call(
        _lin1_body,
        out_shape=[jax.ShapeDtypeStruct((P8, 128), f32)] * 4,
    )(x8, w1lo, w1hi, w1rlo, w1rhi)

    parts_lo = _seg_sum()(src, dst, plo)
    parts_hi = _seg_sum()(src, dst, phi)

    q8, s8 = pl.pallas_call(
        _mid_body,
        out_shape=[jax.ShapeDtypeStruct((P8, 128), f32)] * 2,
    )(parts_lo, parts_hi, rlo, rhi, b1lo, b1hi, wlo, whi, wrlo, wrhi)

    parts2 = _seg_sum()(src, dst, q8)

    out = pl.pallas_call(
        _loss_body,
        out_shape=jax.ShapeDtypeStruct((1, 1), f32),
    )(parts2, s8, b2_8, y_exp)

    return (out[0, 0],)


# speedup vs baseline: 1.0498x; 1.0177x over previous
import functools

import jax
import jax.numpy as jnp
from jax import lax
from jax.experimental import pallas as pl
from jax.experimental.pallas import tpu as pltpu
from jax.experimental.pallas import tpu_sc as plsc

N = 10000
D = 128
H = 32
C = 10
E = 320000

NC = 2
NS = 16
NW = NC * NS

CHUNK = 128
NCHUNK = 80
E_PAD = NW * NCHUNK * CHUNK
N_PAD = 10240
LPT = N_PAD // NS
W16 = 16
P8 = N_PAD // 8
PPT = P8 // NS
NV = N // 8


def _dotT(a, w):
    return lax.dot_general(a, w, (((1,), (1,)), ((), ())),
                           preferred_element_type=jnp.float32)


def _dot(a, w):
    return lax.dot_general(a, w, (((1,), (0,)), ((), ())),
                           preferred_element_type=jnp.float32)


def _lin1_body(x8_ref, wlo_ref, whi_ref, plo_ref, phi_ref):
    x8 = x8_ref[...]
    plo_ref[...] = _dot(x8, wlo_ref[...])
    phi_ref[...] = _dot(x8, whi_ref[...])


def _mid_body(plo_ref, phi_ref, rlo_ref, rhi_ref, b1lo_ref, b1hi_ref,
              wlo_ref, whi_ref, wrlo_ref, wrhi_ref, q_ref, s_ref):
    row = lax.broadcasted_iota(jnp.int32, (P8, 128), 0)
    valid = row < NV
    hlo = jnp.maximum(plo_ref[0] + plo_ref[1] + b1lo_ref[...] + rlo_ref[...], 0.0)
    hhi = jnp.maximum(phi_ref[0] + phi_ref[1] + b1hi_ref[...] + rhi_ref[...], 0.0)
    hlo = jnp.where(valid, hlo, 0.0)
    hhi = jnp.where(valid, hhi, 0.0)
    q_ref[...] = _dot(hlo, wlo_ref[...]) + _dot(hhi, whi_ref[...])
    s_ref[...] = _dot(hlo, wrlo_ref[...]) + _dot(hhi, wrhi_ref[...])


def _loss_body(parts_ref, s_ref, b2_ref, y_ref, out_ref):
    lg = parts_ref[0] + parts_ref[1] + s_ref[...] + b2_ref[...]
    lane = lax.broadcasted_iota(jnp.int32, lg.shape, 1)
    col = lane % W16
    lg = jnp.where(col < C, lg, -1e30)
    v = lg
    for k in (8, 4, 2, 1):
        shifted = jnp.concatenate(
            [v[:, k:], jnp.full((P8, k), -1e30, jnp.float32)], axis=1)
        v = jnp.maximum(v, shifted)
    lrow = lax.broadcasted_iota(jnp.int32, (128, 128), 0)
    lcol = lax.broadcasted_iota(jnp.int32, (128, 128), 1)
    B = jnp.where((lrow // W16 == lcol // W16) & (lrow % W16 == 0), 1.0, 0.0)
    m = _dot(jnp.where(col == 0, v, 0.0), B.astype(jnp.float32))
    e = jnp.exp(lg - m)
    srow = lax.broadcasted_iota(jnp.int32, (128, 8), 0)
    scol = lax.broadcasted_iota(jnp.int32, (128, 8), 1)
    S = jnp.where(srow // W16 == scol, 1.0, 0.0).astype(jnp.float32)
    se = _dot(e, S)
    lse_m = jnp.log(se)
    pick = jnp.where(col == y_ref[...], lg - m, 0.0)
    picked_m = _dot(pick, S)
    nll = lse_m - picked_m
    rowi = lax.broadcasted_iota(jnp.int32, nll.shape, 0)
    nll = jnp.where(rowi < NV, nll, 0.0)
    out_ref[...] = (jnp.sum(nll) / jnp.float32(N)).reshape(1, 1)


def _make_seg_sum():
    mesh = plsc.VectorSubcoreMesh(core_axis_name="c", subcore_axis_name="s",
                                  num_cores=NC, num_subcores=NS)

    @functools.partial(
        pl.kernel,
        out_type=jax.ShapeDtypeStruct((NC, P8, 128), jnp.float32),
        mesh=mesh,
        scratch_types=[
            pltpu.VMEM((NCHUNK, CHUNK), jnp.int32),
            pltpu.VMEM((NCHUNK, CHUNK), jnp.int32),
            *[pltpu.VMEM((CHUNK, W16), jnp.float32) for _ in range(8)],
            pltpu.VMEM((PPT, 128), jnp.float32),
            pltpu.VMEM((LPT, W16), jnp.float32),
            pltpu.VMEM_SHARED((N_PAD, W16), jnp.float32),
            pltpu.VMEM_SHARED((N_PAD, W16), jnp.float32),
            pltpu.SemaphoreType.DMA,
            pltpu.SemaphoreType.DMA,
        ],
        compiler_params=pltpu.CompilerParams(use_tc_tiling_on_sc=False),
    )
    def seg(src_hbm, dst_hbm, table_hbm, out_hbm,
            src_v, dst_v, r0, r1, r2, r3, r4, r5, r6, r7, pbig, rbig,
            tab, acc, sem_g, sem_s):
        c = lax.axis_index("c")
        s = lax.axis_index("s")
        w = c * NS + s
        rows_a = [r0, r1, r2, r3]
        rows_b = [r4, r5, r6, r7]

        idx_cp = pltpu.async_copy(src_hbm.at[w], src_v, sem_s)
        pltpu.sync_copy(dst_hbm.at[w], dst_v)

        def zstore(i, carry):
            rbig[i, pl.ds(0, 16)] = jnp.zeros((16,), jnp.float32)
            return carry

        lax.fori_loop(0, LPT, zstore, 0)
        pltpu.sync_copy(rbig, acc.at[pl.ds(s * LPT, LPT)])

        pltpu.sync_copy(table_hbm.at[pl.ds(s * PPT, PPT)], pbig)

        def rbody(i, carry):
            for kk in range(8):
                rbig[8 * i + kk, pl.ds(0, 16)] = pbig[i, pl.ds(16 * kk, 16)]
            return carry

        lax.fori_loop(0, PPT, rbody, 0)
        pltpu.sync_copy(rbig, tab.at[pl.ds(s * LPT, LPT)])
        idx_cp.wait()
        plsc.subcore_barrier()

        def gather(j, buf):
            pltpu.async_copy(tab.at[src_v.at[j]], buf, sem_g)

        def gather_wait(j, buf):
            pltpu.make_async_copy(tab.at[src_v.at[j]], buf, sem_g).wait()

        def scat(j, buf):
            pltpu.async_copy(buf, acc.at[dst_v.at[j]], sem_s, add=True)

        def scat_wait(j, buf):
            pltpu.make_async_copy(buf, acc.at[dst_v.at[j]], sem_s).wait()

        for b in range(4):
            gather(b, rows_a[b])

        npair = NCHUNK // 8

        def body(k, carry):
            base = 8 * k
            for b in range(4):
                gather_wait(base + b, rows_a[b])
                scat(base + b, rows_a[b])
            for b in range(4):
                gather(base + 4 + b, rows_b[b])
            for b in range(4):
                gather_wait(base + 4 + b, rows_b[b])
                scat(base + 4 + b, rows_b[b])
            for b in range(4):
                scat_wait(base + b, rows_a[b])

            @pl.when(k < npair - 1)
            def _():
                for b in range(4):
                    gather(base + 8 + b, rows_a[b])

            for b in range(4):
                scat_wait(base + 4 + b, rows_b[b])
            return carry

        lax.fori_loop(0, npair, body, 0)
        plsc.subcore_barrier()

        pltpu.sync_copy(acc.at[pl.ds(s * LPT, LPT)], rbig)

        def wbody(i, carry):
            for kk in range(8):
                pbig[i, pl.ds(16 * kk, 16)] = rbig[8 * i + kk, pl.ds(0, 16)]
            return carry

        lax.fori_loop(0, PPT, wbody, 0)
        pltpu.sync_copy(pbig, out_hbm.at[c].at[pl.ds(s * PPT, PPT)])

    return seg


_seg_sum_cache = {}


def _seg_sum():
    if "k" not in _seg_sum_cache:
        _seg_sum_cache["k"] = _make_seg_sum()
    return _seg_sum_cache["k"]


def kernel(x, edge_index, y, W1_rel, b1_rel, W1_root, W2_rel, b2_rel, W2_root):
    f32 = jnp.float32

    pad_rows = N + (jnp.arange(E_PAD - E, dtype=jnp.int32) % (N_PAD - N))
    src = jnp.concatenate([edge_index[0], pad_rows]).reshape(NW, NCHUNK, CHUNK)
    dst = jnp.concatenate([edge_index[1], pad_rows]).reshape(NW, NCHUNK, CHUNK)
    x8 = jnp.pad(x, ((0, N_PAD - N), (0, 0))).reshape(P8, 8 * D)
    eye8f = jnp.eye(8, dtype=f32)
    w1lo = jnp.kron(eye8f, W1_rel.T[:, :W16])
    w1hi = jnp.kron(eye8f, W1_rel.T[:, W16:])
    w1rlo = jnp.kron(eye8f, W1_root.T[:, :W16])
    w1rhi = jnp.kron(eye8f, W1_root.T[:, W16:])
    w2rel_p = jnp.zeros((W16, H), f32).at[:C].set(W2_rel)
    w2root_p = jnp.zeros((W16, H), f32).at[:C].set(W2_root)
    eye8 = jnp.eye(8, dtype=f32)
    wlo = jnp.kron(eye8, w2rel_p.T[:W16])
    whi = jnp.kron(eye8, w2rel_p.T[W16:])
    wrlo = jnp.kron(eye8, w2root_p.T[:W16])
    wrhi = jnp.kron(eye8, w2root_p.T[W16:])
    b1lo = jnp.tile(b1_rel[:W16], 8).reshape(1, 128)
    b1hi = jnp.tile(b1_rel[W16:], 8).reshape(1, 128)
    b2_8 = jnp.tile(jnp.zeros((W16,), f32).at[:C].set(b2_rel), 8).reshape(1, 128)
    y_pad = jnp.pad(y.astype(jnp.int32), (0, N_PAD - N)).reshape(P8, 8)
    y_exp = jnp.repeat(y_pad, W16, axis=1)

    plo, phi = pl.pallas_call(
        _lin1_body,
        out_shape=[jax.ShapeDtypeStruct((P8, 128), f32)] * 2,
    )(x8, w1lo, w1hi)

    parts_lo = _seg_sum()(src, dst, plo)
    parts_hi = _seg_sum()(src, dst, phi)

    rlo, rhi = pl.pallas_call(
        _lin1_body,
        out_shape=[jax.ShapeDtypeStruct((P8, 128), f32)] * 2,
    )(x8, w1rlo, w1rhi)

    q8, s8 = pl.pallas_call(
        _mid_body,
        out_shape=[jax.ShapeDtypeStruct((P8, 128), f32)] * 2,
    )(parts_lo, parts_hi, rlo, rhi, b1lo, b1hi, wlo, whi, wrlo, wrhi)

    parts2 = _seg_sum()(src, dst, q8)

    out = pl.pallas_call(
        _loss_body,
        out_shape=jax.ShapeDtypeStruct((1, 1), f32),
    )(parts2, s8, b2_8, y_exp)

    return (out[0, 0],)
